# baseline (speedup 1.0000x reference)
import functools

import jax
import jax.numpy as jnp
from jax import lax
from jax.experimental import pallas as pl
from jax.experimental.pallas import tpu as pltpu
from jax.experimental.pallas import tpu_sc as plsc

B = 16384
D = 64
D2 = 2 * D
NG = 4 * B
NC, NS = 2, 16
NW = NC * NS
ROWS_PER_W = NG // NW
CHUNK = 512
N_CHUNK = ROWS_PER_W // CHUNK

_sc_mesh = plsc.VectorSubcoreMesh(core_axis_name="c", subcore_axis_name="s")


@functools.partial(
    pl.kernel,
    mesh=_sc_mesh,
    out_type=jax.ShapeDtypeStruct((NG, D2), jnp.float32),
    scratch_types=[
        pltpu.VMEM((CHUNK,), jnp.int32),
        pltpu.VMEM((CHUNK, D2), jnp.float32),
        pltpu.SemaphoreType.DMA,
    ],
)
def _sc_gather(table_hbm, idx_hbm, out_hbm, idx_v, rows_v, sem):
    wid = lax.axis_index("s") * NC + lax.axis_index("c")
    base = wid * ROWS_PER_W

    @pl.loop(0, N_CHUNK)
    def _(c):
        off = base + c * CHUNK
        pltpu.sync_copy(idx_hbm.at[pl.ds(off, CHUNK)], idx_v)
        pltpu.async_copy(table_hbm.at[idx_v], rows_v, sem).wait()
        pltpu.sync_copy(rows_v, out_hbm.at[pl.ds(off, CHUNK)])


BB = 2048
NB = B // BB


def _half(block, par_col):
    lo = block[:, :D]
    hi = block[:, D:]
    return jnp.where(par_col == 1, hi, lo)


def _loss_body(pos_ref, neg_ref, p0_ref, p1_ref, par_ref, pr_ref, out_ref):
    par = par_ref[...]
    pos = _half(pos_ref[...], par[:, 0:1])
    neg = _half(neg_ref[...], par[:, 1:2])
    p0 = _half(p0_ref[...], par[:, 2:3])
    p1 = _half(p1_ref[...], par[:, 3:4])
    ps = p0 + p1
    pos_n = jnp.sum(jnp.abs(pos - ps), axis=1)
    neg_n = jnp.sum(jnp.abs(neg - ps), axis=1)
    pr = pr_ref[...][:, 0]
    diff = 1.0 + pr * pos_n - neg_n
    part = jnp.sum(jnp.maximum(diff, 0.0))

    @pl.when(pl.program_id(0) == 0)
    def _():
        out_ref[0, 0] = 0.0

    out_ref[0, 0] += part


_loss_call = pl.pallas_call(
    _loss_body,
    grid=(NB,),
    in_specs=[
        pl.BlockSpec((BB, D2), lambda i: (i, 0)),
        pl.BlockSpec((BB, D2), lambda i: (i + NB, 0)),
        pl.BlockSpec((BB, D2), lambda i: (i + 2 * NB, 0)),
        pl.BlockSpec((BB, D2), lambda i: (i + 3 * NB, 0)),
        pl.BlockSpec((BB, 4), lambda i: (i, 0)),
        pl.BlockSpec((BB, 1), lambda i: (i, 0)),
    ],
    out_specs=pl.BlockSpec((1, 1), lambda i: (0, 0),
                           memory_space=pltpu.SMEM),
    out_shape=jax.ShapeDtypeStruct((1, 1), jnp.float32),
)


def kernel(rel, rel_neg, path_rel, pr, relation_emb):
    idx = jnp.concatenate([
        rel.astype(jnp.int32),
        rel_neg.astype(jnp.int32),
        path_rel[:, 0].astype(jnp.int32),
        path_rel[:, 1].astype(jnp.int32),
    ])
    table2 = relation_emb.reshape(-1, D2)
    gathered = _sc_gather(table2, idx >> 1)
    parity = (idx & 1).reshape(4, B).T
    out = _loss_call(gathered, gathered, gathered, gathered,
                     parity, pr.reshape(B, 1))
    return out[0, 0]


# speedup vs baseline: 1.6992x; 1.6992x over previous
import functools

import jax
import jax.numpy as jnp
from jax import lax
from jax.experimental import pallas as pl
from jax.experimental.pallas import tpu as pltpu
from jax.experimental.pallas import tpu_sc as plsc

B = 16384
D = 64
D2 = 2 * D
R = 1000000
NG = 4 * B
NC, NS = 2, 16
NW = NC * NS
ROWS_PER_W = NG // NW
CHUNK = 512
N_CHUNK = ROWS_PER_W // CHUNK

_sc_mesh = plsc.VectorSubcoreMesh(core_axis_name="c", subcore_axis_name="s")


@functools.partial(
    pl.kernel,
    mesh=_sc_mesh,
    out_type=jax.ShapeDtypeStruct((NG, D2), jnp.float32),
    scratch_types=[
        pltpu.VMEM((CHUNK,), jnp.int32),
        pltpu.VMEM((CHUNK, D2), jnp.float32),
        pltpu.SemaphoreType.DMA,
    ],
)
def _sc_gather(table_hbm, idx_hbm, out_hbm, idx_v, rows_v, sem):
    wid = lax.axis_index("s") * NC + lax.axis_index("c")
    base = wid * ROWS_PER_W

    @pl.loop(0, N_CHUNK)
    def _(c):
        off = base + c * CHUNK
        pltpu.sync_copy(idx_hbm.at[pl.ds(off, CHUNK)], idx_v)
        pltpu.async_copy(table_hbm.at[idx_v], rows_v, sem).wait()
        pltpu.sync_copy(rows_v, out_hbm.at[pl.ds(off, CHUNK)])


TW = 4096
NT = (R + TW - 1) // TW


def _tr_body(xt_ref, out_ref):
    x = xt_ref[...]
    out_ref[:, :D] = jnp.transpose(x)


_tr_call = pl.pallas_call(
    _tr_body,
    grid=(NT,),
    in_specs=[pl.BlockSpec((D, TW), lambda i: (0, i))],
    out_specs=pl.BlockSpec((TW, D2), lambda i: (i, 0)),
    out_shape=jax.ShapeDtypeStruct((R, D2), jnp.float32),
)


BB = 2048
NB = B // BB


def _loss_body(pos_ref, neg_ref, p0_ref, p1_ref, pr_ref, out_ref):
    pos = pos_ref[:, :D]
    neg = neg_ref[:, :D]
    ps = p0_ref[:, :D] + p1_ref[:, :D]
    pos_n = jnp.sum(jnp.abs(pos - ps), axis=1)
    neg_n = jnp.sum(jnp.abs(neg - ps), axis=1)
    pr = pr_ref[...][:, 0]
    diff = 1.0 + pr * pos_n - neg_n
    part = jnp.sum(jnp.maximum(diff, 0.0))

    @pl.when(pl.program_id(0) == 0)
    def _():
        out_ref[0, 0] = 0.0

    out_ref[0, 0] += part


_loss_call = pl.pallas_call(
    _loss_body,
    grid=(NB,),
    in_specs=[
        pl.BlockSpec((BB, D2), lambda i: (i, 0)),
        pl.BlockSpec((BB, D2), lambda i: (i + NB, 0)),
        pl.BlockSpec((BB, D2), lambda i: (i + 2 * NB, 0)),
        pl.BlockSpec((BB, D2), lambda i: (i + 3 * NB, 0)),
        pl.BlockSpec((BB, 1), lambda i: (i, 0)),
    ],
    out_specs=pl.BlockSpec((1, 1), lambda i: (0, 0),
                           memory_space=pltpu.SMEM),
    out_shape=jax.ShapeDtypeStruct((1, 1), jnp.float32),
)


def kernel(rel, rel_neg, path_rel, pr, relation_emb):
    idx = jnp.concatenate([
        rel.astype(jnp.int32),
        rel_neg.astype(jnp.int32),
        path_rel[:, 0].astype(jnp.int32),
        path_rel[:, 1].astype(jnp.int32),
    ])
    table2 = _tr_call(relation_emb.T)
    gathered = _sc_gather(table2, idx)
    out = _loss_call(gathered, gathered, gathered, gathered,
                     pr.reshape(B, 1))
    return out[0, 0]
